# baseline (device time: 30393 ns/iter reference)
import jax
import jax.numpy as jnp
from jax import lax
from jax.experimental import pallas as pl
from jax.experimental.pallas import tpu as pltpu

N_DEV = 4
B, SQ, SKV_SHARD, HQ, DH = 2, 256, 256, 4, 64
DMODEL = 512
BLK = 64
PAYL_N = HQ * DH + HQ * 32


def kernel(x, Wq, K_ext, V_ext, Wo):
    def body(x_ref, wq_ref, k_ref, v_ref, wo_ref, out_ref,
             acc_ref, recv_ref, send_sems, recv_sems):
        my_pos = lax.axis_index("i")
        partner0 = my_pos ^ 1
        partner1 = my_pos ^ 3

        barrier_sem = pltpu.get_barrier_semaphore()
        for nbr in (partner0, partner1):
            pl.semaphore_signal(
                barrier_sem, inc=1,
                device_id=(nbr,), device_id_type=pl.DeviceIdType.MESH,
            )
        pl.semaphore_wait(barrier_sem, 2)

        rows = lax.broadcasted_iota(jnp.int32, (SQ, SKV_SHARD), 0) // BLK
        cols = lax.broadcasted_iota(jnp.int32, (SQ, SKV_SHARD), 1) // BLK
        mask = (my_pos * (SKV_SHARD // BLK) + cols) <= rows

        for b in range(B):
            q_b = jnp.dot(x_ref[b], wq_ref[...],
                          preferred_element_type=jnp.float32)
            for h in range(HQ):
                q_bh = q_b[:, h * DH:(h + 1) * DH]
                k_bh = k_ref[b, :, h, :]
                v_bh = v_ref[b, :, h, :]
                scores = jnp.dot(q_bh, k_bh.T,
                                 preferred_element_type=jnp.float32) * 0.125
                w = jnp.where(mask, jnp.exp(scores), 0.0)
                denom = jnp.sum(w, axis=1, keepdims=True)
                ctx = jnp.dot(w, v_bh,
                              preferred_element_type=jnp.float32)
                acc_ref[b, :, h * DH:(h + 1) * DH] = ctx
                acc_ref[b, :, HQ * DH + h * 32:HQ * DH + (h + 1) * 32] = (
                    jnp.broadcast_to(denom, (SQ, 32)))

        for r, partner in enumerate((partner0, partner1)):
            rdma = pltpu.make_async_remote_copy(
                src_ref=acc_ref,
                dst_ref=recv_ref.at[r],
                send_sem=send_sems.at[r],
                recv_sem=recv_sems.at[r],
                device_id=(partner,),
                device_id_type=pl.DeviceIdType.MESH,
            )
            rdma.start()
            rdma.wait()
            acc_ref[...] = acc_ref[...] + recv_ref[r]

        for b in range(B):
            parts = []
            for h in range(HQ):
                ctx_h = acc_ref[b, :, h * DH:(h + 1) * DH]
                den_h = acc_ref[b, :, HQ * DH + h * 32:HQ * DH + h * 32 + 1]
                parts.append(ctx_h / den_h)
            ctx_full = jnp.concatenate(parts, axis=1)
            out_ref[b] = jnp.dot(ctx_full, wo_ref[...],
                                 preferred_element_type=jnp.float32)

    return pl.pallas_call(
        body,
        out_shape=jax.ShapeDtypeStruct((B, SQ, DMODEL), jnp.float32),
        in_specs=[pl.BlockSpec(memory_space=pltpu.VMEM)] * 5,
        out_specs=pl.BlockSpec(memory_space=pltpu.VMEM),
        scratch_shapes=[
            pltpu.VMEM((B, SQ, PAYL_N), jnp.float32),
            pltpu.VMEM((2, B, SQ, PAYL_N), jnp.float32),
            pltpu.SemaphoreType.DMA((2,)),
            pltpu.SemaphoreType.DMA((2,)),
        ],
        compiler_params=pltpu.CompilerParams(collective_id=0),
    )(x, Wq, K_ext, V_ext, Wo)


# device time: 23962 ns/iter; 1.2684x vs baseline; 1.2684x over previous
import jax
import jax.numpy as jnp
from jax import lax
from jax.experimental import pallas as pl
from jax.experimental.pallas import tpu as pltpu

N_DEV = 4
B, SQ, SKV_SHARD, HQ, DH = 2, 256, 256, 4, 64
DMODEL = 512
BLK = 64
PAYL_N = HQ * DH + HQ * 32


def kernel(x, Wq, K_ext, V_ext, Wo):
    def body(x_ref, wq_ref, k_ref, v_ref, wo_ref, out_ref,
             acc_ref, recv_ref, send_sems, recv_sems):
        my_pos = lax.axis_index("i")
        partner0 = my_pos ^ 1
        partner1 = my_pos ^ 3

        rows = lax.broadcasted_iota(jnp.int32, (SQ, SKV_SHARD), 0) // BLK
        cols = lax.broadcasted_iota(jnp.int32, (SQ, SKV_SHARD), 1) // BLK
        mask = (my_pos * (SKV_SHARD // BLK) + cols) <= rows

        for b in range(B):
            q_b = jnp.dot(x_ref[b], wq_ref[...],
                          preferred_element_type=jnp.float32)
            for h in range(HQ):
                q_bh = q_b[:, h * DH:(h + 1) * DH]
                k_bh = k_ref[b, :, h, :]
                v_bh = v_ref[b, :, h, :]
                scores = jnp.dot(q_bh, k_bh.T,
                                 preferred_element_type=jnp.float32) * 0.125
                w = jnp.where(mask, jnp.exp(scores), 0.0)
                denom = jnp.sum(w, axis=1, keepdims=True)
                ctx = jnp.dot(w, v_bh,
                              preferred_element_type=jnp.float32)
                acc_ref[b, :, h * DH:(h + 1) * DH] = ctx
                acc_ref[b, :, HQ * DH + h * 32:HQ * DH + (h + 1) * 32] = (
                    jnp.broadcast_to(denom, (SQ, 32)))

        barrier_sem = pltpu.get_barrier_semaphore()
        for nbr in (partner0, partner1):
            pl.semaphore_signal(
                barrier_sem, inc=1,
                device_id=(nbr,), device_id_type=pl.DeviceIdType.MESH,
            )
        pl.semaphore_wait(barrier_sem, 2)

        NCHUNK = 4
        ROWS = SQ // 2

        def chunk_at(ref, c):
            return ref.at[c // 2, pl.ds((c % 2) * ROWS, ROWS)]

        r0 = []
        for c in range(NCHUNK):
            rd = pltpu.make_async_remote_copy(
                src_ref=chunk_at(acc_ref, c),
                dst_ref=chunk_at(recv_ref.at[0], c),
                send_sem=send_sems.at[0, c],
                recv_sem=recv_sems.at[0, c],
                device_id=(partner0,),
                device_id_type=pl.DeviceIdType.MESH,
            )
            rd.start()
            r0.append(rd)

        r1 = []
        for c in range(NCHUNK):
            b, lo = c // 2, (c % 2) * ROWS
            r0[c].wait()
            acc_ref[b, pl.ds(lo, ROWS), :] = (
                acc_ref[b, pl.ds(lo, ROWS), :]
                + recv_ref[0, b, pl.ds(lo, ROWS), :])
            rd = pltpu.make_async_remote_copy(
                src_ref=chunk_at(acc_ref, c),
                dst_ref=chunk_at(recv_ref.at[1], c),
                send_sem=send_sems.at[1, c],
                recv_sem=recv_sems.at[1, c],
                device_id=(partner1,),
                device_id_type=pl.DeviceIdType.MESH,
            )
            rd.start()
            r1.append(rd)

        for b in range(B):
            for half in range(2):
                c = b * 2 + half
                lo = half * ROWS
                r1[c].wait()
                acc_ref[b, pl.ds(lo, ROWS), :] = (
                    acc_ref[b, pl.ds(lo, ROWS), :]
                    + recv_ref[1, b, pl.ds(lo, ROWS), :])
            parts = []
            for h in range(HQ):
                ctx_h = acc_ref[b, :, h * DH:(h + 1) * DH]
                den_h = acc_ref[b, :, HQ * DH + h * 32:HQ * DH + h * 32 + 1]
                parts.append(ctx_h / den_h)
            ctx_full = jnp.concatenate(parts, axis=1)
            out_ref[b] = jnp.dot(ctx_full, wo_ref[...],
                                 preferred_element_type=jnp.float32)

    return pl.pallas_call(
        body,
        out_shape=jax.ShapeDtypeStruct((B, SQ, DMODEL), jnp.float32),
        in_specs=[pl.BlockSpec(memory_space=pltpu.VMEM)] * 5,
        out_specs=pl.BlockSpec(memory_space=pltpu.VMEM),
        scratch_shapes=[
            pltpu.VMEM((B, SQ, PAYL_N), jnp.float32),
            pltpu.VMEM((2, B, SQ, PAYL_N), jnp.float32),
            pltpu.SemaphoreType.DMA((2, 4)),
            pltpu.SemaphoreType.DMA((2, 4)),
        ],
        compiler_params=pltpu.CompilerParams(collective_id=0),
    )(x, Wq, K_ext, V_ext, Wo)
